# Initial kernel scaffold; baseline (speedup 1.0000x reference)
import jax
import jax.numpy as jnp
from jax.experimental import pallas as pl


def kernel(item_ids, v_sem, v_id, t_sem, t_id):
    raise NotImplementedError("write your pallas kernel here")


# speedup vs baseline: 10.7978x; 10.7978x over previous
import functools

import jax
import jax.numpy as jnp
from jax import lax
from jax.experimental import pallas as pl
from jax.experimental.pallas import tpu as pltpu
from jax.experimental.pallas import tpu_sc as plsc

_NULL = 16
_MODAL = 32
_ROW = 64

_NW = 32
_CW = 128


def _fuse_body(vs_ref, vi_ref, ts_ref, ti_ref, out_ref):
    vs = vs_ref[...]
    vi = vi_ref[...]
    ts = ts_ref[...]
    ti = ti_ref[...]
    out_ref[...] = jnp.concatenate(
        [vs[:, :_NULL], vs[:, _NULL:] + vi, ts[:, :_NULL], ts[:, _NULL:] + ti],
        axis=1,
    )


def _build_fused(v_sem, v_id, t_sem, t_id):
    n = v_sem.shape[0]
    r = 2000
    return pl.pallas_call(
        _fuse_body,
        grid=(n // r,),
        in_specs=[
            pl.BlockSpec((r, _MODAL), lambda i: (i, 0)),
            pl.BlockSpec((r, _NULL), lambda i: (i, 0)),
            pl.BlockSpec((r, _MODAL), lambda i: (i, 0)),
            pl.BlockSpec((r, _NULL), lambda i: (i, 0)),
        ],
        out_specs=pl.BlockSpec((r, _ROW), lambda i: (i, 0)),
        out_shape=jax.ShapeDtypeStruct((n, _ROW), jnp.float32),
    )(v_sem, v_id, t_sem, t_id)


def _gather_rows(fused, idx3):
    nw, ch, cw = idx3.shape
    total = nw * ch * cw
    mesh = plsc.VectorSubcoreMesh(core_axis_name="c", subcore_axis_name="s")

    @functools.partial(
        pl.kernel,
        mesh=mesh,
        compiler_params=pltpu.CompilerParams(use_tc_tiling_on_sc=False),
        out_type=jax.ShapeDtypeStruct((total, _ROW), jnp.float32),
        scratch_types=[
            pltpu.VMEM((ch, cw), jnp.int32),
            pltpu.VMEM((cw, _ROW), jnp.float32),
            pltpu.SemaphoreType.DMA,
        ],
    )
    def k(fused_hbm, idx_hbm, out_hbm, idx_v, rows_v, sem):
        wid = lax.axis_index("s") * 2 + lax.axis_index("c")
        pltpu.sync_copy(idx_hbm.at[wid], idx_v)
        base = wid * (ch * cw)

        def body(j, carry):
            pltpu.async_copy(fused_hbm.at[idx_v.at[j]], rows_v, sem).wait()
            pltpu.sync_copy(rows_v, out_hbm.at[pl.ds(base + j * cw, cw)])
            return carry

        lax.fori_loop(0, ch, body, 0)

    return k(fused, idx3)


def kernel(item_ids, v_sem, v_id, t_sem, t_id):
    batch, hist = item_ids.shape
    fused = _build_fused(v_sem, v_id, t_sem, t_id)
    total = batch * hist
    ch = total // (_NW * _CW)
    idx3 = item_ids.reshape(_NW, ch, _CW).astype(jnp.int32)
    out = _gather_rows(fused, idx3)
    return out.reshape(batch, hist, _ROW)


# speedup vs baseline: 12.5380x; 1.1612x over previous
import functools

import jax
import jax.numpy as jnp
from jax import lax
from jax.experimental import pallas as pl
from jax.experimental.pallas import tpu as pltpu
from jax.experimental.pallas import tpu_sc as plsc

_NULL = 16
_MODAL = 32
_ROW = 64

_NW = 32
_CW = 128


def _fuse_body(vs_ref, vi_ref, ts_ref, ti_ref, out_ref):
    vs = vs_ref[...]
    vi = vi_ref[...]
    ts = ts_ref[...]
    ti = ti_ref[...]
    out_ref[...] = jnp.concatenate(
        [vs[:, :_NULL], vs[:, _NULL:] + vi, ts[:, :_NULL], ts[:, _NULL:] + ti],
        axis=1,
    )


def _build_fused(v_sem, v_id, t_sem, t_id):
    n = v_sem.shape[0]
    r = 2000
    return pl.pallas_call(
        _fuse_body,
        grid=(n // r,),
        in_specs=[
            pl.BlockSpec((r, _MODAL), lambda i: (i, 0)),
            pl.BlockSpec((r, _NULL), lambda i: (i, 0)),
            pl.BlockSpec((r, _MODAL), lambda i: (i, 0)),
            pl.BlockSpec((r, _NULL), lambda i: (i, 0)),
        ],
        out_specs=pl.BlockSpec((r, _ROW), lambda i: (i, 0)),
        out_shape=jax.ShapeDtypeStruct((n, _ROW), jnp.float32),
    )(v_sem, v_id, t_sem, t_id)


_NBUF = 4


def _gather_rows(fused, idx3):
    nw, ch, cw = idx3.shape
    total = nw * ch * cw
    mesh = plsc.VectorSubcoreMesh(core_axis_name="c", subcore_axis_name="s")

    @functools.partial(
        pl.kernel,
        mesh=mesh,
        compiler_params=pltpu.CompilerParams(use_tc_tiling_on_sc=False),
        out_type=jax.ShapeDtypeStruct((total, _ROW), jnp.float32),
        scratch_types=(
            [pltpu.VMEM((ch, cw), jnp.int32)]
            + [pltpu.VMEM((cw, _ROW), jnp.float32)] * _NBUF
            + [pltpu.SemaphoreType.DMA] * (2 * _NBUF)
        ),
    )
    def k(fused_hbm, idx_hbm, out_hbm, idx_v, *bufs):
        rows = bufs[:_NBUF]
        gs = bufs[_NBUF:2 * _NBUF]
        ws = bufs[2 * _NBUF:]
        wid = lax.axis_index("s") * 2 + lax.axis_index("c")
        pltpu.sync_copy(idx_hbm.at[wid], idx_v)
        base = wid * (ch * cw)

        def start_gather(b, j):
            pltpu.async_copy(fused_hbm.at[idx_v.at[j]], rows[b], gs[b])

        def wait_gather(b):
            pltpu.make_async_copy(fused_hbm.at[idx_v.at[0]], rows[b], gs[b]).wait()

        def start_write(b, j):
            pltpu.async_copy(rows[b], out_hbm.at[pl.ds(base + j * cw, cw)], ws[b])

        def wait_write(b):
            pltpu.make_async_copy(rows[b], out_hbm.at[pl.ds(base, cw)], ws[b]).wait()

        for b in range(_NBUF):
            start_gather(b, b)

        def outer(i0, carry):
            for b in range(_NBUF):
                j = i0 * _NBUF + b
                wait_gather(b)
                start_write(b, j)
                wait_write(b)
                start_gather(b, j + _NBUF)
            return carry

        lax.fori_loop(0, ch // _NBUF - 1, outer, 0)

        for b in range(_NBUF):
            j = ch - _NBUF + b
            wait_gather(b)
            start_write(b, j)
        for b in range(_NBUF):
            wait_write(b)

    return k(fused, idx3)


def kernel(item_ids, v_sem, v_id, t_sem, t_id):
    batch, hist = item_ids.shape
    fused = _build_fused(v_sem, v_id, t_sem, t_id)
    total = batch * hist
    ch = total // (_NW * _CW)
    idx3 = item_ids.reshape(_NW, ch, _CW).astype(jnp.int32)
    out = _gather_rows(fused, idx3)
    return out.reshape(batch, hist, _ROW)
